# Initial kernel scaffold; baseline (speedup 1.0000x reference)
import jax
import jax.numpy as jnp
from jax.experimental import pallas as pl


def kernel(user_table, item_table, edge_index):
    raise NotImplementedError("write your pallas kernel here")


# speedup vs baseline: 8.3638x; 8.3638x over previous
import functools

import jax
import jax.numpy as jnp
from jax import lax
from jax.experimental import pallas as pl
from jax.experimental.pallas import tpu as pltpu
from jax.experimental.pallas import tpu_sc as plsc

N_USERS = 25000
N = 50000
D = 64
E = 800000
H = 32
DEGW = 16

NC = 2
NT = 16
B = 128
CPB = 8
CHUNKS = 400
BLOCKS = CHUNKS // CPB
EP_T = CHUNKS * B
PAD = NT * EP_T - E
SCRAP = N
NA = 51200
ZR = NA // NT
WB = 3128
RR = 2000

_mesh = plsc.VectorSubcoreMesh(core_axis_name="c", subcore_axis_name="s")


def _zero_fill(buf, width):
    def body(i, _):
        for w in range(width // 16):
            buf[i, pl.ds(w * 16, 16)] = jnp.zeros((16,), jnp.float32)
        return 0
    lax.fori_loop(0, B, body, 0)


def _zero_accum(accum, zbuf, t):
    def body(k, _):
        pltpu.sync_copy(zbuf, accum.at[pl.ds(t * ZR + k * B, B)])
        return 0
    lax.fori_loop(0, ZR // B, body, 0)


def _writeback(accum, out_hbm, c, t):
    @pl.when(t < NT - 1)
    def _():
        pltpu.sync_copy(accum.at[pl.ds(t * WB, WB)],
                        out_hbm.at[c, pl.ds(t * WB, WB)])

    @pl.when(t == NT - 1)
    def _():
        pltpu.sync_copy(accum.at[pl.ds((NT - 1) * WB, N - (NT - 1) * WB)],
                        out_hbm.at[c, pl.ds((NT - 1) * WB, N - (NT - 1) * WB)])


@functools.partial(
    pl.kernel,
    out_type=jax.ShapeDtypeStruct((NC, N, H), jnp.float32),
    mesh=_mesh,
    compiler_params=pltpu.CompilerParams(use_tc_tiling_on_sc=False),
    scratch_types=[
        pltpu.VMEM((CPB, B), jnp.int32),
        pltpu.VMEM((CPB, B), jnp.int32),
        pltpu.VMEM((B, H), jnp.float32),
        pltpu.VMEM((B, H), jnp.float32),
        pltpu.VMEM_SHARED((NA, H), jnp.float32),
    ],
)
def _agg(s_hbm, col_hbm, row_hbm, out_hbm, col_v, row_v, buf, zbuf, accum):
    c = lax.axis_index("c")
    t = lax.axis_index("s")
    _zero_fill(zbuf, H)
    _zero_accum(accum, zbuf, t)
    plsc.subcore_barrier()

    def block(b, _):
        pltpu.sync_copy(col_hbm.at[c, t, pl.ds(b * CPB, CPB)], col_v)
        pltpu.sync_copy(row_hbm.at[t, pl.ds(b * CPB, CPB)], row_v)
        for j in range(CPB):
            pltpu.sync_copy(s_hbm.at[col_v.at[j]], buf)
            pltpu.sync_copy(buf, accum.at[row_v.at[j]], add=True)
        return 0
    lax.fori_loop(0, BLOCKS, block, 0)
    plsc.subcore_barrier()
    _writeback(accum, out_hbm, c, t)


@functools.partial(
    pl.kernel,
    out_type=jax.ShapeDtypeStruct((NC, N, DEGW), jnp.float32),
    mesh=_mesh,
    compiler_params=pltpu.CompilerParams(use_tc_tiling_on_sc=False),
    scratch_types=[
        pltpu.VMEM((CPB, B), jnp.int32),
        pltpu.VMEM((B, DEGW), jnp.float32),
        pltpu.VMEM((B, DEGW), jnp.float32),
        pltpu.VMEM_SHARED((NA, DEGW), jnp.float32),
    ],
)
def _deg(row_hbm, out_hbm, row_v, ones_v, zbuf, accum):
    c = lax.axis_index("c")
    t = lax.axis_index("s")

    def fill(i, _):
        ones_v[i, pl.ds(0, 16)] = jnp.ones((16,), jnp.float32)
        zbuf[i, pl.ds(0, 16)] = jnp.zeros((16,), jnp.float32)
        return 0
    lax.fori_loop(0, B, fill, 0)
    _zero_accum(accum, zbuf, t)
    plsc.subcore_barrier()

    half = BLOCKS // NC

    def block(b, _):
        pltpu.sync_copy(row_hbm.at[t, pl.ds(b * CPB, CPB)], row_v)
        for j in range(CPB):
            pltpu.sync_copy(ones_v, accum.at[row_v.at[j]], add=True)
        return 0
    lax.fori_loop(c * half, (c + 1) * half, block, 0)
    plsc.subcore_barrier()
    _writeback(accum, out_hbm, c, t)


def _dis_body(degp_ref, dis_ref, dis2_ref):
    d = degp_ref[0] + degp_ref[1]
    dis = jnp.where(d > 0, lax.rsqrt(d), 0.0)
    dis_ref[...] = dis
    dis2_ref[...] = dis * dis


_dis_call = pl.pallas_call(
    _dis_body,
    grid=(N // RR,),
    in_specs=[pl.BlockSpec((NC, RR, DEGW), lambda i: (0, i, 0))],
    out_specs=[pl.BlockSpec((RR, DEGW), lambda i: (i, 0)),
               pl.BlockSpec((RR, DEGW), lambda i: (i, 0))],
    out_shape=[jax.ShapeDtypeStruct((N, DEGW), jnp.float32)] * 2,
)


def _scale0_body(e_ref, dis_ref, s_ref):
    dv = dis_ref[:, 0:1]
    s_ref[0] = e_ref[:, 0:H] * dv
    s_ref[1] = e_ref[:, H:D] * dv


_scale0_call = pl.pallas_call(
    _scale0_body,
    grid=(N // RR,),
    in_specs=[pl.BlockSpec((RR, D), lambda i: (i, 0)),
              pl.BlockSpec((RR, DEGW), lambda i: (i, 0))],
    out_specs=pl.BlockSpec((NC, RR, H), lambda i: (0, i, 0)),
    out_shape=jax.ShapeDtypeStruct((NC, N, H), jnp.float32),
)


def _scalek_body(z_ref, dis2_ref, s_ref):
    dv = dis2_ref[:, 0:1][None]
    s_ref[...] = z_ref[...] * dv


_scalek_call = pl.pallas_call(
    _scalek_body,
    grid=(N // RR,),
    in_specs=[pl.BlockSpec((NC, RR, H), lambda i: (0, i, 0)),
              pl.BlockSpec((RR, DEGW), lambda i: (i, 0))],
    out_specs=pl.BlockSpec((NC, RR, H), lambda i: (0, i, 0)),
    out_shape=jax.ShapeDtypeStruct((NC, N, H), jnp.float32),
)


def _mean_body(e0_ref, z1_ref, z2_ref, z3_ref, dis_ref, out_ref):
    zs = z1_ref[...] + z2_ref[...] + z3_ref[...]
    dv = dis_ref[:, 0:1]
    out_ref[:, 0:H] = 0.25 * (e0_ref[:, 0:H] + dv * zs[0])
    out_ref[:, H:D] = 0.25 * (e0_ref[:, H:D] + dv * zs[1])


_mean_call = pl.pallas_call(
    _mean_body,
    grid=(N // RR,),
    in_specs=[pl.BlockSpec((RR, D), lambda i: (i, 0)),
              pl.BlockSpec((NC, RR, H), lambda i: (0, i, 0)),
              pl.BlockSpec((NC, RR, H), lambda i: (0, i, 0)),
              pl.BlockSpec((NC, RR, H), lambda i: (0, i, 0)),
              pl.BlockSpec((RR, DEGW), lambda i: (i, 0))],
    out_specs=pl.BlockSpec((RR, D), lambda i: (i, 0)),
    out_shape=jax.ShapeDtypeStruct((N, D), jnp.float32),
)


@jax.jit
def _run(user_table, item_table, edge_index):
    e0 = jnp.concatenate([user_table, item_table], axis=0)
    ei = edge_index.astype(jnp.int32)
    rowp = jnp.concatenate(
        [ei[0], jnp.full((PAD,), SCRAP, jnp.int32)]).reshape(NT, CHUNKS, B)
    colp = jnp.concatenate(
        [ei[1], jnp.zeros((PAD,), jnp.int32)]).reshape(NT, CHUNKS, B)
    col2 = jnp.stack([colp, colp + N])
    col_deg = jnp.concatenate(
        [ei[1], jnp.full((PAD,), SCRAP, jnp.int32)]).reshape(NT, CHUNKS, B)

    degp = _deg(col_deg)
    dis, dis2 = _dis_call(degp)

    s = _scale0_call(e0, dis).reshape(NC * N, H)
    z1 = _agg(s, col2, rowp)
    s = _scalek_call(z1, dis2).reshape(NC * N, H)
    z2 = _agg(s, col2, rowp)
    s = _scalek_call(z2, dis2).reshape(NC * N, H)
    z3 = _agg(s, col2, rowp)

    out = _mean_call(e0, z1, z2, z3, dis)
    return out[:N_USERS], out[N_USERS:]


def kernel(user_table, item_table, edge_index):
    return _run(user_table, item_table, edge_index)


# speedup vs baseline: 9.1221x; 1.0907x over previous
import functools

import jax
import jax.numpy as jnp
from jax import lax
from jax.experimental import pallas as pl
from jax.experimental.pallas import tpu as pltpu
from jax.experimental.pallas import tpu_sc as plsc

N_USERS = 25000
N = 50000
D = 64
E = 800000
H = 32
DEGW = 16

NC = 2
NT = 16
B = 128
CPB = 8
CHUNKS = 400
BLOCKS = CHUNKS // CPB
EP_T = CHUNKS * B
PAD = NT * EP_T - E
SCRAP = N
NA = 51200
ZR = NA // NT
WB = 3128
RR = 2000

_mesh = plsc.VectorSubcoreMesh(core_axis_name="c", subcore_axis_name="s")


def _zero_fill(buf, width):
    def body(i, _):
        for w in range(width // 16):
            buf[i, pl.ds(w * 16, 16)] = jnp.zeros((16,), jnp.float32)
        return 0
    lax.fori_loop(0, B, body, 0)


def _zero_accum(accum, zbuf, t):
    def body(k, _):
        pltpu.sync_copy(zbuf, accum.at[pl.ds(t * ZR + k * B, B)])
        return 0
    lax.fori_loop(0, ZR // B, body, 0)


def _writeback(accum, out_hbm, c, t):
    @pl.when(t < NT - 1)
    def _():
        pltpu.sync_copy(accum.at[pl.ds(t * WB, WB)],
                        out_hbm.at[c, pl.ds(t * WB, WB)])

    @pl.when(t == NT - 1)
    def _():
        pltpu.sync_copy(accum.at[pl.ds((NT - 1) * WB, N - (NT - 1) * WB)],
                        out_hbm.at[c, pl.ds((NT - 1) * WB, N - (NT - 1) * WB)])


@functools.partial(
    pl.kernel,
    out_type=jax.ShapeDtypeStruct((NC, N, H), jnp.float32),
    mesh=_mesh,
    compiler_params=pltpu.CompilerParams(use_tc_tiling_on_sc=False),
    scratch_types=[
        pltpu.VMEM((CPB, B), jnp.int32),
        pltpu.VMEM((CPB, B), jnp.int32),
        pltpu.VMEM((B, H), jnp.float32),
        pltpu.VMEM((B, H), jnp.float32),
        pltpu.VMEM((B, H), jnp.float32),
        pltpu.VMEM_SHARED((NA, H), jnp.float32),
        pltpu.SemaphoreType.DMA,
    ],
)
def _agg(s_hbm, col_hbm, row_hbm, out_hbm, col_v, row_v, buf0, buf1, zbuf,
         accum, gsem):
    c = lax.axis_index("c")
    t = lax.axis_index("s")
    _zero_fill(zbuf, H)
    _zero_accum(accum, zbuf, t)
    plsc.subcore_barrier()

    bufs = (buf0, buf1)

    def block(b, _):
        pltpu.sync_copy(col_hbm.at[c, t, pl.ds(b * CPB, CPB)], col_v)
        pltpu.sync_copy(row_hbm.at[t, pl.ds(b * CPB, CPB)], row_v)
        d = pltpu.async_copy(s_hbm.at[col_v.at[0]], buf0, gsem)
        for j in range(CPB):
            d.wait()
            if j + 1 < CPB:
                d = pltpu.async_copy(s_hbm.at[col_v.at[j + 1]],
                                     bufs[(j + 1) % 2], gsem)
            pltpu.sync_copy(bufs[j % 2], accum.at[row_v.at[j]], add=True)
        return 0
    lax.fori_loop(0, BLOCKS, block, 0)
    plsc.subcore_barrier()
    _writeback(accum, out_hbm, c, t)


@functools.partial(
    pl.kernel,
    out_type=jax.ShapeDtypeStruct((NC, N, DEGW), jnp.float32),
    mesh=_mesh,
    compiler_params=pltpu.CompilerParams(use_tc_tiling_on_sc=False),
    scratch_types=[
        pltpu.VMEM((CPB, B), jnp.int32),
        pltpu.VMEM((B, DEGW), jnp.float32),
        pltpu.VMEM((B, DEGW), jnp.float32),
        pltpu.VMEM_SHARED((NA, DEGW), jnp.float32),
        pltpu.SemaphoreType.DMA,
    ],
)
def _deg(row_hbm, out_hbm, row_v, ones_v, zbuf, accum, ssem):
    c = lax.axis_index("c")
    t = lax.axis_index("s")

    def fill(i, _):
        ones_v[i, pl.ds(0, 16)] = jnp.ones((16,), jnp.float32)
        zbuf[i, pl.ds(0, 16)] = jnp.zeros((16,), jnp.float32)
        return 0
    lax.fori_loop(0, B, fill, 0)
    _zero_accum(accum, zbuf, t)
    plsc.subcore_barrier()

    half = BLOCKS // NC

    def block(b, _):
        pltpu.sync_copy(row_hbm.at[t, pl.ds(b * CPB, CPB)], row_v)
        ds = [pltpu.async_copy(ones_v, accum.at[row_v.at[j]], ssem, add=True)
              for j in range(CPB)]
        for d in ds:
            d.wait()
        return 0
    lax.fori_loop(c * half, (c + 1) * half, block, 0)
    plsc.subcore_barrier()
    _writeback(accum, out_hbm, c, t)


def _dis_body(degp_ref, dis_ref, dis2_ref):
    d = degp_ref[0] + degp_ref[1]
    dis = jnp.where(d > 0, lax.rsqrt(d), 0.0)
    dis_ref[...] = dis
    dis2_ref[...] = dis * dis


_dis_call = pl.pallas_call(
    _dis_body,
    grid=(N // RR,),
    in_specs=[pl.BlockSpec((NC, RR, DEGW), lambda i: (0, i, 0))],
    out_specs=[pl.BlockSpec((RR, DEGW), lambda i: (i, 0)),
               pl.BlockSpec((RR, DEGW), lambda i: (i, 0))],
    out_shape=[jax.ShapeDtypeStruct((N, DEGW), jnp.float32)] * 2,
)


def _scale0_body(e_ref, dis_ref, s_ref):
    dv = dis_ref[:, 0:1]
    s_ref[0] = e_ref[:, 0:H] * dv
    s_ref[1] = e_ref[:, H:D] * dv


_scale0_call = pl.pallas_call(
    _scale0_body,
    grid=(N // RR,),
    in_specs=[pl.BlockSpec((RR, D), lambda i: (i, 0)),
              pl.BlockSpec((RR, DEGW), lambda i: (i, 0))],
    out_specs=pl.BlockSpec((NC, RR, H), lambda i: (0, i, 0)),
    out_shape=jax.ShapeDtypeStruct((NC, N, H), jnp.float32),
)


def _scalek_body(z_ref, dis2_ref, s_ref):
    dv = dis2_ref[:, 0:1][None]
    s_ref[...] = z_ref[...] * dv


_scalek_call = pl.pallas_call(
    _scalek_body,
    grid=(N // RR,),
    in_specs=[pl.BlockSpec((NC, RR, H), lambda i: (0, i, 0)),
              pl.BlockSpec((RR, DEGW), lambda i: (i, 0))],
    out_specs=pl.BlockSpec((NC, RR, H), lambda i: (0, i, 0)),
    out_shape=jax.ShapeDtypeStruct((NC, N, H), jnp.float32),
)


def _mean_body(e0_ref, z1_ref, z2_ref, z3_ref, dis_ref, out_ref):
    zs = z1_ref[...] + z2_ref[...] + z3_ref[...]
    dv = dis_ref[:, 0:1]
    out_ref[:, 0:H] = 0.25 * (e0_ref[:, 0:H] + dv * zs[0])
    out_ref[:, H:D] = 0.25 * (e0_ref[:, H:D] + dv * zs[1])


_mean_call = pl.pallas_call(
    _mean_body,
    grid=(N // RR,),
    in_specs=[pl.BlockSpec((RR, D), lambda i: (i, 0)),
              pl.BlockSpec((NC, RR, H), lambda i: (0, i, 0)),
              pl.BlockSpec((NC, RR, H), lambda i: (0, i, 0)),
              pl.BlockSpec((NC, RR, H), lambda i: (0, i, 0)),
              pl.BlockSpec((RR, DEGW), lambda i: (i, 0))],
    out_specs=pl.BlockSpec((RR, D), lambda i: (i, 0)),
    out_shape=jax.ShapeDtypeStruct((N, D), jnp.float32),
)


@jax.jit
def _run(user_table, item_table, edge_index):
    e0 = jnp.concatenate([user_table, item_table], axis=0)
    ei = edge_index.astype(jnp.int32)
    rowp = jnp.concatenate(
        [ei[0], jnp.full((PAD,), SCRAP, jnp.int32)]).reshape(NT, CHUNKS, B)
    colp = jnp.concatenate(
        [ei[1], jnp.zeros((PAD,), jnp.int32)]).reshape(NT, CHUNKS, B)
    col2 = jnp.stack([colp, colp + N])
    col_deg = jnp.concatenate(
        [ei[1], jnp.full((PAD,), SCRAP, jnp.int32)]).reshape(NT, CHUNKS, B)

    degp = _deg(col_deg)
    dis, dis2 = _dis_call(degp)

    s = _scale0_call(e0, dis).reshape(NC * N, H)
    z1 = _agg(s, col2, rowp)
    s = _scalek_call(z1, dis2).reshape(NC * N, H)
    z2 = _agg(s, col2, rowp)
    s = _scalek_call(z2, dis2).reshape(NC * N, H)
    z3 = _agg(s, col2, rowp)

    out = _mean_call(e0, z1, z2, z3, dis)
    return out[:N_USERS], out[N_USERS:]


def kernel(user_table, item_table, edge_index):
    return _run(user_table, item_table, edge_index)


# speedup vs baseline: 10.8259x; 1.1868x over previous
import functools

import jax
import jax.numpy as jnp
from jax import lax
from jax.experimental import pallas as pl
from jax.experimental.pallas import tpu as pltpu
from jax.experimental.pallas import tpu_sc as plsc

N_USERS = 25000
N = 50000
D = 64
E = 800000
H = 32
DEGW = 16

NC = 2
NT = 16
B = 128
CPB = 8
CHUNKS = 400
BLOCKS = CHUNKS // CPB
EP_T = CHUNKS * B
PAD = NT * EP_T - E
SCRAP = N
NA = 51200
ZR = NA // NT
WB = 3128
RR = 2000

_mesh = plsc.VectorSubcoreMesh(core_axis_name="c", subcore_axis_name="s")


def _zero_fill(buf, width):
    def body(i, _):
        for w in range(width // 16):
            buf[i, pl.ds(w * 16, 16)] = jnp.zeros((16,), jnp.float32)
        return 0
    lax.fori_loop(0, B, body, 0)


def _zero_accum(accum, zbuf, t):
    def body(k, _):
        pltpu.sync_copy(zbuf, accum.at[pl.ds(t * ZR + k * B, B)])
        return 0
    lax.fori_loop(0, ZR // B, body, 0)


def _writeback(accum, out_hbm, c, t):
    @pl.when(t < NT - 1)
    def _():
        pltpu.sync_copy(accum.at[pl.ds(t * WB, WB)],
                        out_hbm.at[c, pl.ds(t * WB, WB)])

    @pl.when(t == NT - 1)
    def _():
        pltpu.sync_copy(accum.at[pl.ds((NT - 1) * WB, N - (NT - 1) * WB)],
                        out_hbm.at[c, pl.ds((NT - 1) * WB, N - (NT - 1) * WB)])


@functools.partial(
    pl.kernel,
    out_type=jax.ShapeDtypeStruct((NC, N, H), jnp.float32),
    mesh=_mesh,
    compiler_params=pltpu.CompilerParams(use_tc_tiling_on_sc=False),
    scratch_types=[
        pltpu.VMEM((CPB, B), jnp.int32),
        pltpu.VMEM((CPB, B), jnp.int32),
        pltpu.VMEM((B, H), jnp.float32),
        pltpu.VMEM((B, H), jnp.float32),
        pltpu.VMEM((B, H), jnp.float32),
        pltpu.VMEM((B, H), jnp.float32),
        pltpu.VMEM_SHARED((NA, H), jnp.float32),
        pltpu.SemaphoreType.DMA,
        pltpu.SemaphoreType.DMA,
        pltpu.SemaphoreType.DMA,
    ],
)
def _agg(s_hbm, col_hbm, row_hbm, out_hbm, col_v, row_v, buf0, buf1, buf2,
         zbuf, accum, gsem0, gsem1, ssem):
    c = lax.axis_index("c")
    t = lax.axis_index("s")
    _zero_fill(zbuf, H)
    _zero_accum(accum, zbuf, t)
    plsc.subcore_barrier()

    bufs = (buf0, buf1, buf2)
    gsems = (gsem0, gsem1)

    def block(b, _):
        pltpu.sync_copy(col_hbm.at[c, t, pl.ds(b * CPB, CPB)], col_v)
        pltpu.sync_copy(row_hbm.at[t, pl.ds(b * CPB, CPB)], row_v)
        gds = [None] * CPB
        sds = [None] * CPB
        gds[0] = pltpu.async_copy(s_hbm.at[col_v.at[0]], bufs[0], gsems[0])
        gds[1] = pltpu.async_copy(s_hbm.at[col_v.at[1]], bufs[1], gsems[1])
        for j in range(CPB):
            gds[j].wait()
            if j + 2 < CPB:
                if j >= 1:
                    sds[j - 1].wait()
                gds[j + 2] = pltpu.async_copy(
                    s_hbm.at[col_v.at[j + 2]], bufs[(j + 2) % 3],
                    gsems[j % 2])
            sds[j] = pltpu.async_copy(
                bufs[j % 3], accum.at[row_v.at[j]], ssem, add=True)
        for j in range(CPB - 3, CPB):
            sds[j].wait()
        return 0
    lax.fori_loop(0, BLOCKS, block, 0)
    plsc.subcore_barrier()
    _writeback(accum, out_hbm, c, t)


@functools.partial(
    pl.kernel,
    out_type=jax.ShapeDtypeStruct((NC, N, DEGW), jnp.float32),
    mesh=_mesh,
    compiler_params=pltpu.CompilerParams(use_tc_tiling_on_sc=False),
    scratch_types=[
        pltpu.VMEM((CPB, B), jnp.int32),
        pltpu.VMEM((B, DEGW), jnp.float32),
        pltpu.VMEM((B, DEGW), jnp.float32),
        pltpu.VMEM_SHARED((NA, DEGW), jnp.float32),
        pltpu.SemaphoreType.DMA,
    ],
)
def _deg(row_hbm, out_hbm, row_v, ones_v, zbuf, accum, ssem):
    c = lax.axis_index("c")
    t = lax.axis_index("s")

    def fill(i, _):
        ones_v[i, pl.ds(0, 16)] = jnp.ones((16,), jnp.float32)
        zbuf[i, pl.ds(0, 16)] = jnp.zeros((16,), jnp.float32)
        return 0
    lax.fori_loop(0, B, fill, 0)
    _zero_accum(accum, zbuf, t)
    plsc.subcore_barrier()

    half = BLOCKS // NC

    def block(b, _):
        pltpu.sync_copy(row_hbm.at[t, pl.ds(b * CPB, CPB)], row_v)
        ds = [pltpu.async_copy(ones_v, accum.at[row_v.at[j]], ssem, add=True)
              for j in range(CPB)]
        for d in ds:
            d.wait()
        return 0
    lax.fori_loop(c * half, (c + 1) * half, block, 0)
    plsc.subcore_barrier()
    _writeback(accum, out_hbm, c, t)


def _dis_body(degp_ref, dis_ref, dis2_ref):
    d = degp_ref[0] + degp_ref[1]
    dis = jnp.where(d > 0, lax.rsqrt(d), 0.0)
    dis_ref[...] = dis
    dis2_ref[...] = dis * dis


_dis_call = pl.pallas_call(
    _dis_body,
    grid=(N // RR,),
    in_specs=[pl.BlockSpec((NC, RR, DEGW), lambda i: (0, i, 0))],
    out_specs=[pl.BlockSpec((RR, DEGW), lambda i: (i, 0)),
               pl.BlockSpec((RR, DEGW), lambda i: (i, 0))],
    out_shape=[jax.ShapeDtypeStruct((N, DEGW), jnp.float32)] * 2,
)


def _scale0_body(e_ref, dis_ref, s_ref):
    dv = dis_ref[:, 0:1]
    s_ref[0] = e_ref[:, 0:H] * dv
    s_ref[1] = e_ref[:, H:D] * dv


_scale0_call = pl.pallas_call(
    _scale0_body,
    grid=(N // RR,),
    in_specs=[pl.BlockSpec((RR, D), lambda i: (i, 0)),
              pl.BlockSpec((RR, DEGW), lambda i: (i, 0))],
    out_specs=pl.BlockSpec((NC, RR, H), lambda i: (0, i, 0)),
    out_shape=jax.ShapeDtypeStruct((NC, N, H), jnp.float32),
)


def _scalek_body(z_ref, dis2_ref, s_ref):
    dv = dis2_ref[:, 0:1][None]
    s_ref[...] = z_ref[...] * dv


_scalek_call = pl.pallas_call(
    _scalek_body,
    grid=(N // RR,),
    in_specs=[pl.BlockSpec((NC, RR, H), lambda i: (0, i, 0)),
              pl.BlockSpec((RR, DEGW), lambda i: (i, 0))],
    out_specs=pl.BlockSpec((NC, RR, H), lambda i: (0, i, 0)),
    out_shape=jax.ShapeDtypeStruct((NC, N, H), jnp.float32),
)


def _mean_body(e0_ref, z1_ref, z2_ref, z3_ref, dis_ref, out_ref):
    zs = z1_ref[...] + z2_ref[...] + z3_ref[...]
    dv = dis_ref[:, 0:1]
    out_ref[:, 0:H] = 0.25 * (e0_ref[:, 0:H] + dv * zs[0])
    out_ref[:, H:D] = 0.25 * (e0_ref[:, H:D] + dv * zs[1])


_mean_call = pl.pallas_call(
    _mean_body,
    grid=(N // RR,),
    in_specs=[pl.BlockSpec((RR, D), lambda i: (i, 0)),
              pl.BlockSpec((NC, RR, H), lambda i: (0, i, 0)),
              pl.BlockSpec((NC, RR, H), lambda i: (0, i, 0)),
              pl.BlockSpec((NC, RR, H), lambda i: (0, i, 0)),
              pl.BlockSpec((RR, DEGW), lambda i: (i, 0))],
    out_specs=pl.BlockSpec((RR, D), lambda i: (i, 0)),
    out_shape=jax.ShapeDtypeStruct((N, D), jnp.float32),
)


@jax.jit
def _run(user_table, item_table, edge_index):
    e0 = jnp.concatenate([user_table, item_table], axis=0)
    ei = edge_index.astype(jnp.int32)
    rowp = jnp.concatenate(
        [ei[0], jnp.full((PAD,), SCRAP, jnp.int32)]).reshape(NT, CHUNKS, B)
    colp = jnp.concatenate(
        [ei[1], jnp.zeros((PAD,), jnp.int32)]).reshape(NT, CHUNKS, B)
    col2 = jnp.stack([colp, colp + N])
    col_deg = jnp.concatenate(
        [ei[1], jnp.full((PAD,), SCRAP, jnp.int32)]).reshape(NT, CHUNKS, B)

    degp = _deg(col_deg)
    dis, dis2 = _dis_call(degp)

    s = _scale0_call(e0, dis).reshape(NC * N, H)
    z1 = _agg(s, col2, rowp)
    s = _scalek_call(z1, dis2).reshape(NC * N, H)
    z2 = _agg(s, col2, rowp)
    s = _scalek_call(z2, dis2).reshape(NC * N, H)
    z3 = _agg(s, col2, rowp)

    out = _mean_call(e0, z1, z2, z3, dis)
    return out[:N_USERS], out[N_USERS:]


def kernel(user_table, item_table, edge_index):
    return _run(user_table, item_table, edge_index)


# speedup vs baseline: 11.3744x; 1.0507x over previous
import functools

import jax
import jax.numpy as jnp
from jax import lax
from jax.experimental import pallas as pl
from jax.experimental.pallas import tpu as pltpu
from jax.experimental.pallas import tpu_sc as plsc

N_USERS = 25000
N = 50000
D = 64
E = 800000
H = 32
DEGW = 16

NC = 2
NT = 16
B = 128
CPB = 10
CHUNKS = 400
BLOCKS = CHUNKS // CPB
EP_T = CHUNKS * B
PAD = NT * EP_T - E
SCRAP = N
NA = 51200
ZR = NA // NT
WB = 3128
RR = 2000

_mesh = plsc.VectorSubcoreMesh(core_axis_name="c", subcore_axis_name="s")


def _zero_fill(buf, width):
    def body(i, _):
        for w in range(width // 16):
            buf[i, pl.ds(w * 16, 16)] = jnp.zeros((16,), jnp.float32)
        return 0
    lax.fori_loop(0, B, body, 0)


def _zero_accum(accum, zbuf, t):
    def body(k, _):
        pltpu.sync_copy(zbuf, accum.at[pl.ds(t * ZR + k * B, B)])
        return 0
    lax.fori_loop(0, ZR // B, body, 0)


def _writeback(accum, out_hbm, c, t):
    @pl.when(t < NT - 1)
    def _():
        pltpu.sync_copy(accum.at[pl.ds(t * WB, WB)],
                        out_hbm.at[c, pl.ds(t * WB, WB)])

    @pl.when(t == NT - 1)
    def _():
        pltpu.sync_copy(accum.at[pl.ds((NT - 1) * WB, N - (NT - 1) * WB)],
                        out_hbm.at[c, pl.ds((NT - 1) * WB, N - (NT - 1) * WB)])


@functools.partial(
    pl.kernel,
    out_type=jax.ShapeDtypeStruct((NC, N, H), jnp.float32),
    mesh=_mesh,
    compiler_params=pltpu.CompilerParams(use_tc_tiling_on_sc=False),
    scratch_types=[
        pltpu.VMEM((CPB, B), jnp.int32),
        pltpu.VMEM((CPB, B), jnp.int32),
        pltpu.VMEM((B, H), jnp.float32),
        pltpu.VMEM((B, H), jnp.float32),
        pltpu.VMEM((B, H), jnp.float32),
        pltpu.VMEM((B, H), jnp.float32),
        pltpu.VMEM((B, H), jnp.float32),
        pltpu.VMEM((B, H), jnp.float32),
        pltpu.VMEM_SHARED((NA, H), jnp.float32),
        pltpu.SemaphoreType.DMA,
        pltpu.SemaphoreType.DMA,
        pltpu.SemaphoreType.DMA,
        pltpu.SemaphoreType.DMA,
        pltpu.SemaphoreType.DMA,
    ],
)
def _agg(s_hbm, col_hbm, row_hbm, out_hbm, col_v, row_v, buf0, buf1, buf2,
         buf3, buf4, zbuf, accum, gsem0, gsem1, gsem2, ssem0, ssem1):
    c = lax.axis_index("c")
    t = lax.axis_index("s")
    _zero_fill(zbuf, H)
    _zero_accum(accum, zbuf, t)
    plsc.subcore_barrier()

    bufs = (buf0, buf1, buf2, buf3, buf4)
    gsems = (gsem0, gsem1, gsem2)
    ssems = (ssem0, ssem1)

    def block(b, _):
        pltpu.sync_copy(col_hbm.at[c, t, pl.ds(b * CPB, CPB)], col_v)
        pltpu.sync_copy(row_hbm.at[t, pl.ds(b * CPB, CPB)], row_v)
        gds = [None] * CPB
        sds = [None] * CPB
        for p in range(3):
            gds[p] = pltpu.async_copy(s_hbm.at[col_v.at[p]], bufs[p],
                                      gsems[p])
        for j in range(CPB):
            gds[j].wait()
            if j + 3 < CPB:
                if j >= 2:
                    sds[j - 2].wait()
                gds[j + 3] = pltpu.async_copy(
                    s_hbm.at[col_v.at[j + 3]], bufs[(j + 3) % 5],
                    gsems[(j + 3) % 3])
            sds[j] = pltpu.async_copy(
                bufs[j % 5], accum.at[row_v.at[j]], ssems[j % 2], add=True)
        for j in range(CPB - 5, CPB):
            sds[j].wait()
        return 0
    lax.fori_loop(0, BLOCKS, block, 0)
    plsc.subcore_barrier()
    _writeback(accum, out_hbm, c, t)


@functools.partial(
    pl.kernel,
    out_type=jax.ShapeDtypeStruct((NC, N, DEGW), jnp.float32),
    mesh=_mesh,
    compiler_params=pltpu.CompilerParams(use_tc_tiling_on_sc=False),
    scratch_types=[
        pltpu.VMEM((CPB, B), jnp.int32),
        pltpu.VMEM((B, DEGW), jnp.float32),
        pltpu.VMEM((B, DEGW), jnp.float32),
        pltpu.VMEM_SHARED((NA, DEGW), jnp.float32),
        pltpu.SemaphoreType.DMA,
    ],
)
def _deg(row_hbm, out_hbm, row_v, ones_v, zbuf, accum, ssem):
    c = lax.axis_index("c")
    t = lax.axis_index("s")

    def fill(i, _):
        ones_v[i, pl.ds(0, 16)] = jnp.ones((16,), jnp.float32)
        zbuf[i, pl.ds(0, 16)] = jnp.zeros((16,), jnp.float32)
        return 0
    lax.fori_loop(0, B, fill, 0)
    _zero_accum(accum, zbuf, t)
    plsc.subcore_barrier()

    half = BLOCKS // NC

    def block(b, _):
        pltpu.sync_copy(row_hbm.at[t, pl.ds(b * CPB, CPB)], row_v)
        ds = [pltpu.async_copy(ones_v, accum.at[row_v.at[j]], ssem, add=True)
              for j in range(CPB)]
        for d in ds:
            d.wait()
        return 0
    lax.fori_loop(c * half, (c + 1) * half, block, 0)
    plsc.subcore_barrier()
    _writeback(accum, out_hbm, c, t)


def _dis_body(degp_ref, dis_ref, dis2_ref):
    d = degp_ref[0] + degp_ref[1]
    dis = jnp.where(d > 0, lax.rsqrt(d), 0.0)
    dis_ref[...] = dis
    dis2_ref[...] = dis * dis


_dis_call = pl.pallas_call(
    _dis_body,
    grid=(N // RR,),
    in_specs=[pl.BlockSpec((NC, RR, DEGW), lambda i: (0, i, 0))],
    out_specs=[pl.BlockSpec((RR, DEGW), lambda i: (i, 0)),
               pl.BlockSpec((RR, DEGW), lambda i: (i, 0))],
    out_shape=[jax.ShapeDtypeStruct((N, DEGW), jnp.float32)] * 2,
)


def _scale0_body(e_ref, dis_ref, s_ref):
    dv = dis_ref[:, 0:1]
    s_ref[0] = e_ref[:, 0:H] * dv
    s_ref[1] = e_ref[:, H:D] * dv


_scale0_call = pl.pallas_call(
    _scale0_body,
    grid=(N // RR,),
    in_specs=[pl.BlockSpec((RR, D), lambda i: (i, 0)),
              pl.BlockSpec((RR, DEGW), lambda i: (i, 0))],
    out_specs=pl.BlockSpec((NC, RR, H), lambda i: (0, i, 0)),
    out_shape=jax.ShapeDtypeStruct((NC, N, H), jnp.float32),
)


def _scalek_body(z_ref, dis2_ref, s_ref):
    dv = dis2_ref[:, 0:1][None]
    s_ref[...] = z_ref[...] * dv


_scalek_call = pl.pallas_call(
    _scalek_body,
    grid=(N // RR,),
    in_specs=[pl.BlockSpec((NC, RR, H), lambda i: (0, i, 0)),
              pl.BlockSpec((RR, DEGW), lambda i: (i, 0))],
    out_specs=pl.BlockSpec((NC, RR, H), lambda i: (0, i, 0)),
    out_shape=jax.ShapeDtypeStruct((NC, N, H), jnp.float32),
)


def _mean_body(e0_ref, z1_ref, z2_ref, z3_ref, dis_ref, out_ref):
    zs = z1_ref[...] + z2_ref[...] + z3_ref[...]
    dv = dis_ref[:, 0:1]
    out_ref[:, 0:H] = 0.25 * (e0_ref[:, 0:H] + dv * zs[0])
    out_ref[:, H:D] = 0.25 * (e0_ref[:, H:D] + dv * zs[1])


_mean_call = pl.pallas_call(
    _mean_body,
    grid=(N // RR,),
    in_specs=[pl.BlockSpec((RR, D), lambda i: (i, 0)),
              pl.BlockSpec((NC, RR, H), lambda i: (0, i, 0)),
              pl.BlockSpec((NC, RR, H), lambda i: (0, i, 0)),
              pl.BlockSpec((NC, RR, H), lambda i: (0, i, 0)),
              pl.BlockSpec((RR, DEGW), lambda i: (i, 0))],
    out_specs=pl.BlockSpec((RR, D), lambda i: (i, 0)),
    out_shape=jax.ShapeDtypeStruct((N, D), jnp.float32),
)


@jax.jit
def _run(user_table, item_table, edge_index):
    e0 = jnp.concatenate([user_table, item_table], axis=0)
    ei = edge_index.astype(jnp.int32)
    rowp = jnp.concatenate(
        [ei[0], jnp.full((PAD,), SCRAP, jnp.int32)]).reshape(NT, CHUNKS, B)
    colp = jnp.concatenate(
        [ei[1], jnp.zeros((PAD,), jnp.int32)]).reshape(NT, CHUNKS, B)
    col2 = jnp.stack([colp, colp + N])
    col_deg = jnp.concatenate(
        [ei[1], jnp.full((PAD,), SCRAP, jnp.int32)]).reshape(NT, CHUNKS, B)

    degp = _deg(col_deg)
    dis, dis2 = _dis_call(degp)

    s = _scale0_call(e0, dis).reshape(NC * N, H)
    z1 = _agg(s, col2, rowp)
    s = _scalek_call(z1, dis2).reshape(NC * N, H)
    z2 = _agg(s, col2, rowp)
    s = _scalek_call(z2, dis2).reshape(NC * N, H)
    z3 = _agg(s, col2, rowp)

    out = _mean_call(e0, z1, z2, z3, dis)
    return out[:N_USERS], out[N_USERS:]


def kernel(user_table, item_table, edge_index):
    return _run(user_table, item_table, edge_index)
